# baseline (speedup 1.0000x reference)
import functools

import jax
import jax.numpy as jnp
from jax import lax
from jax.experimental import pallas as pl
from jax.experimental.pallas import tpu as pltpu
from jax.experimental.pallas import tpu_sc as plsc

_E = 128
_V = 1001
_B = 4
_N = 512
_K = 16
_D = 2 * _E

_NC, _NS, _L = 2, 16, 16
_NW = _NC * _NS
_ROWS = _B * _N
_RPW = _ROWS // _NW
_NFH = 2
_NQ = 4
_FH = _D // _NFH
_NQN = _N // _NQ
_NG = _NQN // _L

_F32 = jnp.float32
_HIGH = jax.lax.Precision.HIGHEST


def _sc_mesh():
    return plsc.VectorSubcoreMesh(core_axis_name="c", subcore_axis_name="s",
                                  num_cores=_NC, num_subcores=_NS)


def _dot(x, w):
    return jax.lax.dot_general(
        x, w, (((x.ndim - 1,), (0,)), ((), ())),
        preferred_element_type=_F32, precision=_HIGH)


def _small_linear(x, w, b):
    s = None
    for d in range(w.shape[0]):
        t = x[:, d:d + 1] * w[d:d + 1, :]
        s = t if s is None else s + t
    return s + b


def _mlp_chain(x, layers):
    n = len(layers)
    for i, (w, b) in enumerate(layers):
        x = _dot(x, w) + b
        if i < n - 1:
            x = jnp.maximum(x, 0.0)
    return x


def _wid():
    return lax.axis_index("s") * _NC + lax.axis_index("c")


@functools.cache
def _make_sc_class_gather():
    @functools.partial(
        pl.kernel,
        out_type=jax.ShapeDtypeStruct((_ROWS, _E), _F32),
        scratch_types=[pltpu.VMEM((_RPW,), jnp.int32),
                       pltpu.VMEM((_RPW, _E), _F32),
                       pltpu.SemaphoreType.DMA],
        mesh=_sc_mesh())
    def _sc_class_gather(table_hbm, idx_hbm, out_hbm, idx_v, rows_v, sem):
        base = _wid() * _RPW
        pltpu.sync_copy(idx_hbm.at[pl.ds(base, _RPW)], idx_v)
        pltpu.async_copy(table_hbm.at[idx_v], rows_v, sem).wait()
        pltpu.sync_copy(rows_v, out_hbm.at[pl.ds(base, _RPW)])
    return _sc_class_gather


@functools.cache
def _make_sc_gather_max():
    @functools.partial(
        pl.kernel,
        out_type=jax.ShapeDtypeStruct((_NW, _NQN * _FH), _F32),
        scratch_types=[pltpu.VMEM((_K, _NQN), jnp.int32),
                       pltpu.VMEM((_N * _FH,), _F32),
                       pltpu.VMEM((_NQN * _FH,), _F32)],
        compiler_params=pltpu.CompilerParams(needs_layout_passes=False),
        mesh=_sc_mesh())
    def _sc_gather_max(bbp_hbm, idxT_hbm, gp_hbm, idx_v, bb_v, g_v):
        w = _wid()
        b = w // (_NFH * _NQ)
        fh = (w // _NQ) % _NFH
        q = w % _NQ
        pltpu.sync_copy(idxT_hbm.at[b, :, pl.ds(q * _NQN, _NQN)], idx_v)
        pltpu.sync_copy(bbp_hbm.at[b * _NFH + fh], bb_v)

        lane = lax.iota(jnp.int32, _L)

        def group_body(gi, carry):
            gsl = pl.ds(pl.multiple_of(gi * _L, _L), _L)
            rowb = [idx_v[k, gsl] * _FH for k in range(_K)]
            srowb = (gi * _L + lane) * _FH
            for f in range(_FH):
                acc = plsc.load_gather(bb_v, [rowb[0] + f])
                for k in range(1, _K):
                    acc = jnp.maximum(acc,
                                      plsc.load_gather(bb_v, [rowb[k] + f]))
                plsc.store_scatter(g_v, [srowb + f], acc)
            return carry

        lax.fori_loop(0, _NG, group_body, 0, unroll=False)
        pltpu.sync_copy(g_v, gp_hbm.at[w])
    return _sc_gather_max


def _reassemble_g(gp_ref):
    qs = []
    for q in range(_NQ):
        qs.append(jnp.concatenate([gp_ref[0, 0, q], gp_ref[0, 1, q]], axis=1))
    return jnp.concatenate(qs, axis=0)


def _knn_stage(x, ew, eb):
    w1 = ew[:_D, :]
    w2 = ew[_D:, :]
    a = _dot(x, w1 - w2) + eb
    bb = _dot(x, w2)

    gram = jax.lax.dot_general(
        x, x, (((1,), (1,)), ((), ())),
        preferred_element_type=_F32, precision=_HIGH)
    iota_j = jax.lax.broadcasted_iota(jnp.int32, (_N, _N), 1)
    iota_i = jax.lax.broadcasted_iota(jnp.int32, (_N, _N), 0)
    sq_col = jnp.sum(x * x, axis=1, keepdims=True)
    sq_row = jnp.sum(jnp.where(iota_i == iota_j, gram, 0.0),
                     axis=0, keepdims=True)
    dist = sq_col + sq_row - 2.0 * gram

    iota_row_f = jax.lax.broadcasted_iota(jnp.int32, (1, _N), 1).astype(_F32)
    rows = []
    inf = jnp.float32(jnp.inf)
    for _ in range(_K):
        m = jnp.min(dist, axis=1, keepdims=True)
        cand = jnp.where(dist <= m, iota_j, _N)
        jmin = jnp.min(cand, axis=1, keepdims=True)
        onehot = iota_j == jmin
        r = jax.lax.dot_general(
            iota_row_f, jnp.where(onehot, 1.0, 0.0),
            (((1,), (1,)), ((), ())),
            preferred_element_type=_F32, precision=_HIGH)
        rows.append(r)
        dist = jnp.where(onehot, inf, dist)
    idx_t = jnp.concatenate(rows, axis=0).astype(jnp.int32)
    return a, bb, idx_t


def _stage_a_body(cemb_ref, pos_ref, desc_ref,
                  pw0, pb0, pw1, pb1, pw2, pb2, pw3, pb3, ew, eb,
                  a_out, bb_out, idx_out):
    p = pos_ref[0]
    p = jnp.maximum(_small_linear(p, pw0[...], pb0[...]), 0.0)
    p = jnp.maximum(_small_linear(p, pw1[...], pb1[...]), 0.0)
    p = jnp.maximum(_small_linear(p, pw2[...], pb2[...]), 0.0)
    pos_emb = _small_linear(p, pw3[...], pb3[...])
    desc_b = jnp.broadcast_to(desc_ref[0], (_N, _E))
    x = jnp.concatenate([cemb_ref[0] + pos_emb, desc_b], axis=1)
    a, bb, idx_t = _knn_stage(x, ew[...], eb[...])
    a_out[0] = a
    bb_out[0, 0] = bb[:, :_FH]
    bb_out[0, 1] = bb[:, _FH:]
    idx_out[0] = idx_t


def _stage_c_body(a_ref, gp_ref, ew, eb, a_out, bb_out, idx_out, x_out):
    x = a_ref[0] + _reassemble_g(gp_ref)
    a, bb, idx_t = _knn_stage(x, ew[...], eb[...])
    a_out[0] = a
    bb_out[0, 0] = bb[:, :_FH]
    bb_out[0, 1] = bb[:, _FH:]
    idx_out[0] = idx_t
    x_out[0] = x


def _stage_e_body(x1_ref, a1_ref, gp1_ref, desc_ref,
                  fw, fb,
                  rw0, rb0, rw1, rb1, rw2, rb2, rw3, rb3,
                  tw0, tb0, tw1, tb1,
                  cw0, cb0, cw1, cb1, cw2, cb2,
                  ow0, ob0, ow1, ob1, ow2, ob2,
                  feats_out, ref_out, tcls_out, ocls_out, ooff_out):
    desc = desc_ref[0]
    desc_b = jnp.broadcast_to(desc, (_N, _E))
    x2 = a1_ref[0] + _reassemble_g(gp1_ref)
    cat = jnp.concatenate([x1_ref[0], x2, desc_b], axis=1)
    feats = _dot(cat, fw[...]) + fb[...]
    feats_out[0] = feats
    ref_out[0] = _mlp_chain(
        feats, [(rw0[...], rb0[...]), (rw1[...], rb1[...]),
                (rw2[...], rb2[...]), (rw3[...], rb3[...])])
    tcls_out[0] = _mlp_chain(
        desc, [(tw0[...], tb0[...]), (tw1[...], tb1[...])])
    ocls_out[0] = _mlp_chain(
        feats, [(cw0[...], cb0[...]), (cw1[...], cb1[...]),
                (cw2[...], cb2[...])])
    ooff_out[0] = _mlp_chain(
        feats, [(ow0[...], ob0[...]), (ow1[...], ob1[...]),
                (ow2[...], ob2[...])])


def _batch_spec(shape):
    nd = len(shape)
    return pl.BlockSpec((1,) + shape[1:],
                        lambda b, _nd=nd: (b,) + (0,) * (_nd - 1))


def _full_spec(shape):
    nd = len(shape)
    return pl.BlockSpec(shape, lambda b, _nd=nd: (0,) * nd)


def _wb(layers):
    out = []
    for w, b in layers:
        out.append(w)
        out.append(b.reshape(1, -1))
    return out


@jax.jit
def kernel(class_indices, object_positions, description_encodings, params):
    desc3 = description_encodings.reshape(_B, 1, _E)
    cls_flat = class_indices.astype(jnp.int32).reshape(_ROWS)

    cemb = _make_sc_class_gather()(params["class_table"], cls_flat)
    cemb = cemb.reshape(_B, _N, _E)

    pos_w = _wb(params["pos_mlp"])
    arb = pltpu.CompilerParams(dimension_semantics=("arbitrary",))

    abi_out_specs = [_batch_spec((_B, _N, _D)),
                     _batch_spec((_B, _NFH, _N, _FH)),
                     _batch_spec((_B, _K, _N))]
    abi_out_shape = [jax.ShapeDtypeStruct((_B, _N, _D), _F32),
                     jax.ShapeDtypeStruct((_B, _NFH, _N, _FH), _F32),
                     jax.ShapeDtypeStruct((_B, _K, _N), jnp.int32)]
    gp5 = (_B, _NFH, _NQ, _NQN, _FH)

    ew0, eb0 = params["edge_mlps"][0][0]
    a0, bb0, idx0 = pl.pallas_call(
        _stage_a_body,
        grid=(_B,),
        in_specs=[_batch_spec((_B, _N, _E)),
                  _batch_spec((_B, _N, 2)),
                  _batch_spec((_B, 1, _E))]
                 + [_full_spec(w.shape) for w in pos_w]
                 + [_full_spec(ew0.shape), _full_spec((1, _D))],
        out_specs=abi_out_specs,
        out_shape=abi_out_shape,
        compiler_params=arb,
    )(cemb, object_positions, desc3, *pos_w, ew0, eb0.reshape(1, -1))

    gp0 = _make_sc_gather_max()(bb0.reshape(_B * _NFH, _N * _FH), idx0)

    ew1, eb1 = params["edge_mlps"][1][0]
    a1, bb1, idx1, x1 = pl.pallas_call(
        _stage_c_body,
        grid=(_B,),
        in_specs=[_batch_spec((_B, _N, _D)),
                  _batch_spec(gp5),
                  _full_spec(ew1.shape), _full_spec((1, _D))],
        out_specs=abi_out_specs + [_batch_spec((_B, _N, _D))],
        out_shape=abi_out_shape + [jax.ShapeDtypeStruct((_B, _N, _D), _F32)],
        compiler_params=arb,
    )(a0, gp0.reshape(gp5), ew1, eb1.reshape(1, -1))

    gp1 = _make_sc_gather_max()(bb1.reshape(_B * _NFH, _N * _FH), idx1)

    head_w = (_wb(params["mlp_features"]) + _wb(params["mlp_object_ref"])
              + _wb(params["mlp_target_class"])
              + _wb(params["mlp_object_class"])
              + _wb(params["mlp_object_offset"]))
    feats, oref, tcls, ocls, ooff = pl.pallas_call(
        _stage_e_body,
        grid=(_B,),
        in_specs=[_batch_spec((_B, _N, _D)),
                  _batch_spec((_B, _N, _D)),
                  _batch_spec(gp5),
                  _batch_spec((_B, 1, _E))]
                 + [_full_spec(w.shape) for w in head_w],
        out_specs=[_batch_spec((_B, _N, _D)),
                   _batch_spec((_B, _N, 1)),
                   _batch_spec((_B, 1, _V)),
                   _batch_spec((_B, _N, _V)),
                   _batch_spec((_B, _N, 2))],
        out_shape=[jax.ShapeDtypeStruct((_B, _N, _D), _F32),
                   jax.ShapeDtypeStruct((_B, _N, 1), _F32),
                   jax.ShapeDtypeStruct((_B, 1, _V), _F32),
                   jax.ShapeDtypeStruct((_B, _N, _V), _F32),
                   jax.ShapeDtypeStruct((_B, _N, 2), _F32)],
        compiler_params=arb,
    )(x1, a1, gp1.reshape(gp5), desc3, *head_w)

    return (feats, oref[..., 0], tcls[:, 0, :], ocls, ooff)


# speedup vs baseline: 2.3202x; 2.3202x over previous
import functools

import jax
import jax.numpy as jnp
from jax import lax
from jax.experimental import pallas as pl
from jax.experimental.pallas import tpu as pltpu
from jax.experimental.pallas import tpu_sc as plsc

_E = 128
_V = 1001
_B = 4
_N = 512
_K = 16
_D = 2 * _E

_NC, _NS, _L = 2, 16, 16
_NW = _NC * _NS
_ROWS = _B * _N
_RPW = _ROWS // _NW
_CH = 8
_NCHUNK = _RPW // _CH
_CROWS = _CH * _K
_NBUF = 3

_F32 = jnp.float32
_BF16 = jnp.bfloat16


def _sc_mesh():
    return plsc.VectorSubcoreMesh(core_axis_name="c", subcore_axis_name="s",
                                  num_cores=_NC, num_subcores=_NS)


def _dot(x, w):
    return jax.lax.dot_general(
        x.astype(_BF16), w.astype(_BF16), (((x.ndim - 1,), (0,)), ((), ())),
        preferred_element_type=_F32)


def _mlp_chain(x, layers):
    n = len(layers)
    for i, (w, b) in enumerate(layers):
        x = _dot(x, w[...]) + b[...]
        if i < n - 1:
            x = jnp.maximum(x, 0.0)
    return x


def _wid():
    return lax.axis_index("s") * _NC + lax.axis_index("c")


@functools.cache
def _make_sc_class_gather():
    @functools.partial(
        pl.kernel,
        out_type=jax.ShapeDtypeStruct((_ROWS, _E), _F32),
        scratch_types=[pltpu.VMEM((_RPW,), jnp.int32),
                       pltpu.VMEM((_RPW, _E), _F32),
                       pltpu.SemaphoreType.DMA],
        mesh=_sc_mesh())
    def _sc_class_gather(table_hbm, idx_hbm, out_hbm, idx_v, rows_v, sem):
        base = _wid() * _RPW
        pltpu.sync_copy(idx_hbm.at[pl.ds(base, _RPW)], idx_v)
        pltpu.async_copy(table_hbm.at[idx_v], rows_v, sem).wait()
        pltpu.sync_copy(rows_v, out_hbm.at[pl.ds(base, _RPW)])
    return _sc_class_gather


@functools.cache
def _make_sc_gather():
    @functools.partial(
        pl.kernel,
        out_type=jax.ShapeDtypeStruct((_ROWS * _K, _D), _F32),
        scratch_types=[pltpu.VMEM((_RPW * _K,), jnp.int32)]
                      + [pltpu.VMEM((_CROWS, _D), _F32)] * _NBUF
                      + [pltpu.SemaphoreType.DMA] * (2 * _NBUF),
        mesh=_sc_mesh())
    def _sc_gather(x_hbm, idx_hbm, out_hbm, idx_v, *bufsem):
        bufs = bufsem[:_NBUF]
        gsems = bufsem[_NBUF:2 * _NBUF]
        osems = bufsem[2 * _NBUF:]
        base = _wid() * _RPW
        pltpu.sync_copy(idx_hbm.at[pl.ds(base * _K, _RPW * _K)], idx_v)

        def gstart(c, s):
            return pltpu.async_copy(
                x_hbm.at[idx_v.at[pl.ds(c * _CROWS, _CROWS)]],
                bufs[s], gsems[s])

        def ostart(c, s):
            return pltpu.async_copy(
                bufs[s], out_hbm.at[pl.ds(base * _K + c * _CROWS, _CROWS)],
                osems[s])

        gcp = [gstart(c, c) for c in range(_NBUF)]
        ocp = [None] * _NBUF
        for c in range(_NCHUNK):
            s = c % _NBUF
            gcp[s].wait()
            ocp[s] = ostart(c, s)
            if c + _NBUF < _NCHUNK:
                ocp[s].wait()
                gcp[s] = gstart(c + _NBUF, s)
        for s in range(_NBUF):
            ocp[s].wait()
    return _sc_gather


def _knn_select(x):
    xb = x.astype(_BF16)
    gram = jax.lax.dot_general(
        xb, xb, (((1,), (1,)), ((), ())),
        preferred_element_type=_F32)
    iota_j = jax.lax.broadcasted_iota(jnp.int32, (_N, _N), 1)
    iota_i = jax.lax.broadcasted_iota(jnp.int32, (_N, _N), 0)
    sq_col = jnp.sum(x * x, axis=1, keepdims=True)
    sq_row = jnp.sum(jnp.where(iota_i == iota_j, sq_col, 0.0),
                     axis=0, keepdims=True)
    dist = (sq_col + sq_row) - 2.0 * gram

    goff = pl.program_id(0) * _N
    cols = []
    inf = jnp.float32(jnp.inf)
    for _ in range(_K):
        m = jnp.min(dist, axis=1, keepdims=True)
        cand = jnp.where(dist <= m, iota_j, _N)
        jmin = jnp.min(cand, axis=1, keepdims=True)
        onehot = iota_j == jmin
        cols.append(jmin + goff)
        dist = jnp.where(onehot, inf, dist)
    idx = jnp.concatenate(cols, axis=1)
    return idx


def _edge_next(x, xg, a_nob, ew, eb):
    diffs = xg.reshape(_N, _K, _D) - x[:, None, :]
    c = _dot(diffs.reshape(_N * _K, _D), ew[_D:, :])
    cmax = jnp.max(c.reshape(_N, _K, _D), axis=1)
    return (a_nob + cmax) + eb


def _stage_a_body(cemb_ref, pos_ref, desc_ref,
                  pw0, pb0, pw1, pb1, pw2, pb2, pw3, pb3, ew,
                  x_out, a_out, idx_out):
    p = pos_ref[0]
    p = jnp.maximum(_dot(p, pw0[...]) + pb0[...], 0.0)
    p = jnp.maximum(_dot(p, pw1[...]) + pb1[...], 0.0)
    p = jnp.maximum(_dot(p, pw2[...]) + pb2[...], 0.0)
    pos_emb = _dot(p, pw3[...]) + pb3[...]
    desc_b = jnp.broadcast_to(desc_ref[0], (_N, _E))
    x = jnp.concatenate([cemb_ref[0] + pos_emb, desc_b], axis=1)
    x_out[0] = x
    a_out[0] = _dot(x, ew[...][:_D, :])
    idx_out[0] = _knn_select(x)


def _stage_c_body(x0_ref, xg_ref, a0_ref, ew0, eb0, ew1,
                  x_out, a_out, idx_out):
    x = _edge_next(x0_ref[0], xg_ref[0], a0_ref[0], ew0[...], eb0[...])
    x_out[0] = x
    a_out[0] = _dot(x, ew1[...][:_D, :])
    idx_out[0] = _knn_select(x)


def _stage_e_body(x1_ref, xg_ref, a1_ref, desc_ref, ew1, eb1,
                  fw, fb,
                  rw0, rb0, rw1, rb1, rw2, rb2, rw3, rb3,
                  tw0, tb0, tw1, tb1,
                  cw0, cb0, cw1, cb1, cw2, cb2,
                  ow0, ob0, ow1, ob1, ow2, ob2,
                  feats_out, ref_out, tcls_out, ocls_out, ooff_out):
    desc = desc_ref[0]
    desc_b = jnp.broadcast_to(desc, (_N, _E))
    x2 = _edge_next(x1_ref[0], xg_ref[0], a1_ref[0], ew1[...], eb1[...])
    cat = jnp.concatenate([x1_ref[0], x2, desc_b], axis=1)
    feats = _dot(cat, fw[...]) + fb[...]
    feats_out[0] = feats
    ref_out[0] = _mlp_chain(
        feats, [(rw0, rb0), (rw1, rb1), (rw2, rb2), (rw3, rb3)])
    tcls_out[0] = _mlp_chain(desc, [(tw0, tb0), (tw1, tb1)])
    ocls_out[0] = _mlp_chain(feats, [(cw0, cb0), (cw1, cb1), (cw2, cb2)])
    ooff_out[0] = _mlp_chain(feats, [(ow0, ob0), (ow1, ob1), (ow2, ob2)])


def _batch_spec(shape):
    nd = len(shape)
    return pl.BlockSpec((1,) + shape[1:],
                        lambda b, _nd=nd: (b,) + (0,) * (_nd - 1))


def _full_spec(shape):
    nd = len(shape)
    return pl.BlockSpec(shape, lambda b, _nd=nd: (0,) * nd)


def _wb(layers):
    out = []
    for w, b in layers:
        out.append(w)
        out.append(b.reshape(1, -1))
    return out


@jax.jit
def kernel(class_indices, object_positions, description_encodings, params):
    desc3 = description_encodings.reshape(_B, 1, _E)
    cls_flat = class_indices.astype(jnp.int32).reshape(_ROWS)

    cemb = _make_sc_class_gather()(params["class_table"], cls_flat)
    cemb = cemb.reshape(_B, _N, _E)

    pos_w = _wb(params["pos_mlp"])
    arb = pltpu.CompilerParams(dimension_semantics=("arbitrary",))

    xai_specs = [_batch_spec((_B, _N, _D)),
                 _batch_spec((_B, _N, _D)),
                 _batch_spec((_B, _N, _K))]
    xai_shape = [jax.ShapeDtypeStruct((_B, _N, _D), _F32),
                 jax.ShapeDtypeStruct((_B, _N, _D), _F32),
                 jax.ShapeDtypeStruct((_B, _N, _K), jnp.int32)]

    ew0, eb0 = params["edge_mlps"][0][0]
    ew1, eb1 = params["edge_mlps"][1][0]

    x0, a0, idx0 = pl.pallas_call(
        _stage_a_body,
        grid=(_B,),
        in_specs=[_batch_spec((_B, _N, _E)),
                  _batch_spec((_B, _N, 2)),
                  _batch_spec((_B, 1, _E))]
                 + [_full_spec(w.shape) for w in pos_w]
                 + [_full_spec(ew0.shape)],
        out_specs=xai_specs,
        out_shape=xai_shape,
        compiler_params=arb,
    )(cemb, object_positions, desc3, *pos_w, ew0)

    xg0 = _make_sc_gather()(x0.reshape(_ROWS, _D), idx0.reshape(_ROWS * _K))
    xg0 = xg0.reshape(_B, _N * _K, _D)

    x1, a1, idx1 = pl.pallas_call(
        _stage_c_body,
        grid=(_B,),
        in_specs=[_batch_spec((_B, _N, _D)),
                  _batch_spec((_B, _N * _K, _D)),
                  _batch_spec((_B, _N, _D)),
                  _full_spec(ew0.shape), _full_spec((1, _D)),
                  _full_spec(ew1.shape)],
        out_specs=xai_specs,
        out_shape=xai_shape,
        compiler_params=arb,
    )(x0, xg0, a0, ew0, eb0.reshape(1, -1), ew1)

    xg1 = _make_sc_gather()(x1.reshape(_ROWS, _D), idx1.reshape(_ROWS * _K))
    xg1 = xg1.reshape(_B, _N * _K, _D)

    head_w = (_wb(params["mlp_features"]) + _wb(params["mlp_object_ref"])
              + _wb(params["mlp_target_class"])
              + _wb(params["mlp_object_class"])
              + _wb(params["mlp_object_offset"]))
    feats, oref, tcls, ocls, ooff = pl.pallas_call(
        _stage_e_body,
        grid=(_B,),
        in_specs=[_batch_spec((_B, _N, _D)),
                  _batch_spec((_B, _N * _K, _D)),
                  _batch_spec((_B, _N, _D)),
                  _batch_spec((_B, 1, _E)),
                  _full_spec(ew1.shape), _full_spec((1, _D))]
                 + [_full_spec(w.shape) for w in head_w],
        out_specs=[_batch_spec((_B, _N, _D)),
                   _batch_spec((_B, _N, 1)),
                   _batch_spec((_B, 1, _V)),
                   _batch_spec((_B, _N, _V)),
                   _batch_spec((_B, _N, 2))],
        out_shape=[jax.ShapeDtypeStruct((_B, _N, _D), _F32),
                   jax.ShapeDtypeStruct((_B, _N, 1), _F32),
                   jax.ShapeDtypeStruct((_B, 1, _V), _F32),
                   jax.ShapeDtypeStruct((_B, _N, _V), _F32),
                   jax.ShapeDtypeStruct((_B, _N, 2), _F32)],
        compiler_params=arb,
    )(x1, xg1, a1, desc3, ew1, eb1.reshape(1, -1), *head_w)

    return (feats, oref[..., 0], tcls[:, 0, :], ocls, ooff)
